# baseline (speedup 1.0000x reference)
import functools

import jax
import jax.numpy as jnp
from jax import lax
from jax.experimental import pallas as pl
from jax.experimental.pallas import tpu as pltpu
from jax.experimental.pallas import tpu_sc as plsc

_N = 10000
_D = 256
_E = 160000
_NUM_PRED = 5
_DH = 128
_NP = 10112
_NS = 16
_CHUNK = 64
_ET = 10240
_NCHT = _ET // _CHUNK
_NCH = _NS * _NCHT
_EPAD = _NS * _ET
_RPT = _NP // _NS


def _take16(v, idx):
    return lax.gather(
        v, idx.reshape(16, 1),
        lax.GatherDimensionNumbers(offset_dims=(), collapsed_slice_dims=(0,),
                                   start_index_map=(0,)),
        slice_sizes=(1,), mode=lax.GatherScatterMode.PROMISE_IN_BOUNDS)


def _deg_count(idx_ref, deg_ref):
    iota = lax.iota(jnp.int32, 16)
    prev_i = jnp.maximum(iota - 1, 0)
    next_i = jnp.minimum(iota + 1, 15)
    for g in range(_CHUNK // 16):
        d = idx_ref[pl.ds(g * 16, 16)]
        d, _ = plsc.sort_key_val(d, d)
        dprev = _take16(d, prev_i)
        dnext = _take16(d, next_i)
        starts = (iota == 0) | (d != dprev)
        ends = (iota == 15) | (d != dnext)
        st = plsc.cummax(jnp.where(starts, iota, 0))
        cnt = (iota - st + 1).astype(jnp.float32)
        plsc.addupdate_scatter(deg_ref, [d], cnt, mask=ends)


def _sc_agg(xp, gidx, didx, zinit):
    mesh = plsc.VectorSubcoreMesh(core_axis_name="c", subcore_axis_name="s")

    @functools.partial(
        pl.kernel,
        out_type=[jax.ShapeDtypeStruct((2, _NP, _DH), jnp.float32),
                  jax.ShapeDtypeStruct((_NS, _NP), jnp.float32)],
        mesh=mesh,
        scratch_types=[
            [pltpu.VMEM((_CHUNK,), jnp.int32) for _ in range(8)],
            [pltpu.VMEM((_CHUNK,), jnp.int32) for _ in range(8)],
            [pltpu.VMEM((_CHUNK, _DH), jnp.float32) for _ in range(4)],
            pltpu.VMEM((_NP,), jnp.float32),
            pltpu.VMEM_SHARED((_NP, _DH), jnp.float32),
            [pltpu.SemaphoreType.DMA for _ in range(8)],
            [pltpu.SemaphoreType.DMA for _ in range(8)],
            [pltpu.SemaphoreType.DMA for _ in range(4)],
            [pltpu.SemaphoreType.DMA for _ in range(4)],
        ],
        compiler_params=pltpu.CompilerParams(use_tc_tiling_on_sc=True, needs_layout_passes=False),
    )
    def k(xp_hbm, gidx_hbm, didx_hbm, z_hbm, out_hbm, deg_hbm,
          gbuf, dbuf, rows, deg_v, agg_sh, isem, jsem, gsem, ssem):
        c = lax.axis_index("c")
        s = lax.axis_index("s")
        r0 = s * _RPT
        pltpu.sync_copy(z_hbm.at[pl.ds(r0, _RPT)], agg_sh.at[pl.ds(r0, _RPT)])
        on_c0 = c == 0

        @pl.when(on_c0)
        def _():

            def zero_deg(i, carry):
                deg_v[pl.ds(i * 16, 16)] = jnp.zeros((16,), jnp.float32)
                return carry

            lax.fori_loop(0, _NP // 16, zero_deg, 0)

        ebase = (s * _NCHT) * _CHUNK
        gbase = c * _EPAD + ebase

        def load_idx(i, a):
            pltpu.async_copy(
                gidx_hbm.at[pl.ds(gbase + i * _CHUNK, _CHUNK)], gbuf[a], isem[a])
            pltpu.async_copy(
                didx_hbm.at[pl.ds(ebase + i * _CHUNK, _CHUNK)], dbuf[a], jsem[a])

        def wait_idx(i, a):
            pltpu.make_async_copy(
                gidx_hbm.at[pl.ds(gbase + i * _CHUNK, _CHUNK)], gbuf[a], isem[a]).wait()
            pltpu.make_async_copy(
                didx_hbm.at[pl.ds(ebase + i * _CHUNK, _CHUNK)], dbuf[a], jsem[a]).wait()

        plsc.subcore_barrier()
        for a in range(6):
            load_idx(a, a)
        for b in range(2):
            wait_idx(b, b)
            pltpu.async_copy(xp_hbm.at[gbuf[b]], rows[b], gsem[b])

        def outer(g, carry):
            for a in range(8):
                i = g * 8 + a
                b = a % 4
                pltpu.make_async_copy(
                    xp_hbm.at[gbuf[a]], rows[b], gsem[b]).wait()
                pltpu.async_copy(
                    rows[b], agg_sh.at[dbuf[a]], ssem[b], add=True)
                b2 = (b + 2) % 4
                a2 = (a + 2) % 8
                a6 = (a + 6) % 8

                @pl.when(i >= 2)
                def _():
                    pltpu.make_async_copy(
                        rows[b2], agg_sh.at[dbuf[a6]], ssem[b2]).wait()

                @pl.when(i + 6 < _NCHT)
                def _():
                    load_idx(i + 6, a6)

                @pl.when(i + 2 < _NCHT)
                def _():
                    wait_idx(i + 2, a2)
                    pltpu.async_copy(xp_hbm.at[gbuf[a2]], rows[b2], gsem[b2])

                @pl.when(on_c0)
                def _():
                    _deg_count(dbuf[a], deg_v)
            return carry

        lax.fori_loop(0, _NCHT // 8, outer, 0)
        for j in (_NCHT - 2, _NCHT - 1):
            pltpu.make_async_copy(
                rows[j % 4], agg_sh.at[dbuf[j % 8]], ssem[j % 4]).wait()
        plsc.subcore_barrier()
        pltpu.sync_copy(agg_sh.at[pl.ds(r0, _RPT)], out_hbm.at[c, pl.ds(r0, _RPT)])

        @pl.when(on_c0)
        def _():
            pltpu.sync_copy(deg_v, deg_hbm.at[s])

    return k(xp, gidx, didx, zinit)


def _tc_body(x_ref, aA_ref, aB_ref, dg_ref, nz_ref, ws_ref, wnA_ref, wnB_ref,
             be_ref, wd_ref, bd_ref, wf_ref, bf_ref, pm_ref, pf_ref):
    d = jnp.maximum(dg_ref[...], 1.0)
    aA = aA_ref[0] / d
    aB = aB_ref[0] / d
    h = jnp.dot(x_ref[...], ws_ref[...], preferred_element_type=jnp.float32)
    h = h + jnp.dot(aA, wnA_ref[...], preferred_element_type=jnp.float32)
    h = h + jnp.dot(aB, wnB_ref[...], preferred_element_type=jnp.float32)
    h = jnp.maximum(h + be_ref[...], 0.0) + nz_ref[...]
    pm_ref[...] = jnp.maximum(
        jnp.dot(h, wd_ref[...], preferred_element_type=jnp.float32) + bd_ref[...], 0.0)
    pf_ref[...] = jnp.tanh(
        jnp.dot(h, wf_ref[...], preferred_element_type=jnp.float32) + bf_ref[...])


_BN = 1000


def _tc_dense(x, aggout, deg, noise, W_self, WnA, WnB, be2, Wd8, bd8, W_f, bf2):
    return pl.pallas_call(
        _tc_body,
        grid=(_N // _BN,),
        in_specs=[
            pl.BlockSpec((_BN, _D), lambda i: (i, 0)),
            pl.BlockSpec((1, _BN, _DH), lambda i: (0, i, 0)),
            pl.BlockSpec((1, _BN, _DH), lambda i: (1, i, 0)),
            pl.BlockSpec((_BN, 1), lambda i: (i, 0)),
            pl.BlockSpec((_BN, _D), lambda i: (i, 0)),
            pl.BlockSpec((_D, _D), lambda i: (0, 0)),
            pl.BlockSpec((_DH, _D), lambda i: (0, 0)),
            pl.BlockSpec((_DH, _D), lambda i: (0, 0)),
            pl.BlockSpec((1, _D), lambda i: (0, 0)),
            pl.BlockSpec((_D, 8), lambda i: (0, 0)),
            pl.BlockSpec((1, 8), lambda i: (0, 0)),
            pl.BlockSpec((_D, _NUM_PRED * _D), lambda i: (0, 0)),
            pl.BlockSpec((1, _NUM_PRED * _D), lambda i: (0, 0)),
        ],
        out_specs=[
            pl.BlockSpec((_BN, 8), lambda i: (i, 0)),
            pl.BlockSpec((_BN, _NUM_PRED * _D), lambda i: (i, 0)),
        ],
        out_shape=[
            jax.ShapeDtypeStruct((_N, 8), jnp.float32),
            jax.ShapeDtypeStruct((_N, _NUM_PRED * _D), jnp.float32),
        ],
    )(x, aggout, aggout, deg, noise, W_self, WnA, WnB, be2, Wd8, bd8, W_f, bf2)


def kernel(x, edge_index, noise, W_self, W_neigh, b_enc, W_d, b_d, W_f, b_f):
    src = edge_index[0].astype(jnp.int32)
    dst = edge_index[1].astype(jnp.int32)
    xp = x.reshape(2 * _N, _DH)
    npad = _EPAD - _E
    src_p = jnp.concatenate([src, jnp.zeros((npad,), jnp.int32)])
    dst_p = jnp.concatenate([dst, jnp.full((npad,), _N, jnp.int32)])
    gidx = jnp.concatenate([2 * src_p, 2 * src_p + 1])
    zinit = jnp.zeros((_NP, _DH), jnp.float32)

    aggout, deg16 = _sc_agg(xp, gidx, dst_p, zinit)

    be2 = b_enc.reshape(1, _D)
    Wd8 = jnp.pad(W_d, ((0, 0), (0, 7)))
    bd8 = jnp.pad(b_d, (0, 7)).reshape(1, 8)
    bf2 = b_f.reshape(1, _NUM_PRED * _D)
    WnA = W_neigh[:_DH]
    WnB = W_neigh[_DH:]
    deg2 = deg16.sum(axis=0)[:_N].reshape(_N, 1)

    pm8, pf = _tc_dense(x, aggout, deg2, noise, W_self, WnA, WnB,
                        be2, Wd8, bd8, W_f, bf2)
    return (pm8[:, :1], pf.reshape(_N, _NUM_PRED, _D))


# speedup vs baseline: 1.2046x; 1.2046x over previous
import functools

import jax
import jax.numpy as jnp
from jax import lax
from jax.experimental import pallas as pl
from jax.experimental.pallas import tpu as pltpu
from jax.experimental.pallas import tpu_sc as plsc

_N = 10000
_D = 256
_E = 160000
_NUM_PRED = 5
_DH = 128
_NP = 10112
_NS = 16
_CHUNK = 64
_ET = 10240
_NCHT = _ET // _CHUNK
_NCH = _NS * _NCHT
_EPAD = _NS * _ET
_RPT = _NP // _NS


def _take16(v, idx):
    return lax.gather(
        v, idx.reshape(16, 1),
        lax.GatherDimensionNumbers(offset_dims=(), collapsed_slice_dims=(0,),
                                   start_index_map=(0,)),
        slice_sizes=(1,), mode=lax.GatherScatterMode.PROMISE_IN_BOUNDS)


def _deg_count(idx_ref, deg_ref):
    iota = lax.iota(jnp.int32, 16)
    prev_i = jnp.maximum(iota - 1, 0)
    next_i = jnp.minimum(iota + 1, 15)
    for g in range(_CHUNK // 16):
        d = idx_ref[pl.ds(g * 16, 16)]
        d, _ = plsc.sort_key_val(d, d)
        dprev = _take16(d, prev_i)
        dnext = _take16(d, next_i)
        starts = (iota == 0) | (d != dprev)
        ends = (iota == 15) | (d != dnext)
        st = plsc.cummax(jnp.where(starts, iota, 0))
        cnt = (iota - st + 1).astype(jnp.float32)
        plsc.addupdate_scatter(deg_ref, [d], cnt, mask=ends)


def _sc_agg(xp, gidx, didx):
    mesh = plsc.VectorSubcoreMesh(core_axis_name="c", subcore_axis_name="s")

    @functools.partial(
        pl.kernel,
        out_type=[jax.ShapeDtypeStruct((2, _NP, _DH), jnp.float32),
                  jax.ShapeDtypeStruct((2 * _NS, _NP), jnp.float32)],
        mesh=mesh,
        scratch_types=[
            [pltpu.VMEM((_CHUNK,), jnp.int32) for _ in range(8)],
            [pltpu.VMEM((_CHUNK,), jnp.int32) for _ in range(8)],
            [pltpu.VMEM((_CHUNK, _DH), jnp.float32) for _ in range(4)],
            pltpu.VMEM((_NP,), jnp.float32),
            pltpu.VMEM_SHARED((_NP, _DH), jnp.float32),
            [pltpu.SemaphoreType.DMA for _ in range(8)],
            [pltpu.SemaphoreType.DMA for _ in range(8)],
            [pltpu.SemaphoreType.DMA for _ in range(4)],
            [pltpu.SemaphoreType.DMA for _ in range(4)],
        ],
        compiler_params=pltpu.CompilerParams(use_tc_tiling_on_sc=True, needs_layout_passes=False),
    )
    def k(xp_hbm, gidx_hbm, didx_hbm, out_hbm, deg_hbm,
          gbuf, dbuf, rows, deg_v, agg_sh, isem, jsem, gsem, ssem):
        c = lax.axis_index("c")
        s = lax.axis_index("s")
        r0 = s * _RPT

        def zero_rows(i, carry):
            for j in range(_DH // 16):
                rows[0][i, pl.ds(j * 16, 16)] = jnp.zeros((16,), jnp.float32)
            return carry

        lax.fori_loop(0, _CHUNK, zero_rows, 0)
        for q in range(_RPT // _CHUNK):
            pltpu.sync_copy(rows[0], agg_sh.at[pl.ds(r0 + q * _CHUNK, _CHUNK)])
        _REM = _RPT - (_RPT // _CHUNK) * _CHUNK
        if _REM:
            pltpu.sync_copy(
                rows[0].at[pl.ds(0, _REM)],
                agg_sh.at[pl.ds(r0 + _RPT - _REM, _REM)])

        def zero_deg(i, carry):
            deg_v[pl.ds(i * 16, 16)] = jnp.zeros((16,), jnp.float32)
            return carry

        lax.fori_loop(0, _NP // 16, zero_deg, 0)

        ebase = (s * _NCHT) * _CHUNK

        def load_idx(i, a):
            pltpu.async_copy(
                gidx_hbm.at[pl.ds(ebase + i * _CHUNK, _CHUNK)], gbuf[a], isem[a])
            pltpu.async_copy(
                didx_hbm.at[pl.ds(ebase + i * _CHUNK, _CHUNK)], dbuf[a], jsem[a])

        def wait_idx(i, a):
            pltpu.make_async_copy(
                gidx_hbm.at[pl.ds(ebase + i * _CHUNK, _CHUNK)], gbuf[a], isem[a]).wait()
            pltpu.make_async_copy(
                didx_hbm.at[pl.ds(ebase + i * _CHUNK, _CHUNK)], dbuf[a], jsem[a]).wait()
            for j in range(_CHUNK // 16):
                sl = pl.ds(j * 16, 16)
                gbuf[a][sl] = 2 * gbuf[a][sl] + c

        on_c0 = c == 0
        plsc.subcore_barrier()
        for a in range(6):
            load_idx(a, a)
        for b in range(2):
            wait_idx(b, b)
            pltpu.async_copy(xp_hbm.at[gbuf[b]], rows[b], gsem[b])

        def outer(g, carry):
            for a in range(8):
                i = g * 8 + a
                b = a % 4
                pltpu.make_async_copy(
                    xp_hbm.at[gbuf[a]], rows[b], gsem[b]).wait()
                pltpu.async_copy(
                    rows[b], agg_sh.at[dbuf[a]], ssem[b], add=True)
                b2 = (b + 2) % 4
                a2 = (a + 2) % 8
                a6 = (a + 6) % 8

                @pl.when(i >= 2)
                def _():
                    pltpu.make_async_copy(
                        rows[b2], agg_sh.at[dbuf[a6]], ssem[b2]).wait()

                @pl.when(i + 6 < _NCHT)
                def _():
                    load_idx(i + 6, a6)

                @pl.when(i + 2 < _NCHT)
                def _():
                    wait_idx(i + 2, a2)
                    pltpu.async_copy(xp_hbm.at[gbuf[a2]], rows[b2], gsem[b2])

                if a % 2 == 0:
                    @pl.when(on_c0)
                    def _():
                        _deg_count(dbuf[a], deg_v)
                else:
                    @pl.when(jnp.logical_not(on_c0))
                    def _():
                        _deg_count(dbuf[a], deg_v)
            return carry

        lax.fori_loop(0, _NCHT // 8, outer, 0)
        for j in (_NCHT - 2, _NCHT - 1):
            pltpu.make_async_copy(
                rows[j % 4], agg_sh.at[dbuf[j % 8]], ssem[j % 4]).wait()
        plsc.subcore_barrier()
        pltpu.sync_copy(agg_sh.at[pl.ds(r0, _RPT)], out_hbm.at[c, pl.ds(r0, _RPT)])

        pltpu.sync_copy(deg_v, deg_hbm.at[c * _NS + s])

    return k(xp, gidx, didx)


def _tc_pre_body(x_ref, ws_ref, be_ref, hp_ref):
    hp_ref[...] = (jnp.dot(x_ref[...], ws_ref[...],
                           preferred_element_type=jnp.float32) + be_ref[...])


def _tc_pre(x, W_self, be2):
    return pl.pallas_call(
        _tc_pre_body,
        grid=(_N // _BN,),
        in_specs=[
            pl.BlockSpec((_BN, _D), lambda i: (i, 0)),
            pl.BlockSpec((_D, _D), lambda i: (0, 0)),
            pl.BlockSpec((1, _D), lambda i: (0, 0)),
        ],
        out_specs=[pl.BlockSpec((_BN, _D), lambda i: (i, 0))],
        out_shape=[jax.ShapeDtypeStruct((_N, _D), jnp.float32)],
    )(x, W_self, be2)[0]


def _tc_body(hp_ref, aA_ref, aB_ref, dg_ref, nz_ref, wnA_ref, wnB_ref,
             wd_ref, bd_ref, wf_ref, bf_ref, pm_ref, pf_ref):
    d = jnp.maximum(dg_ref[...], 1.0)
    aA = aA_ref[0] / d
    aB = aB_ref[0] / d
    h = hp_ref[...]
    h = h + jnp.dot(aA, wnA_ref[...], preferred_element_type=jnp.float32)
    h = h + jnp.dot(aB, wnB_ref[...], preferred_element_type=jnp.float32)
    h = jnp.maximum(h, 0.0) + nz_ref[...]
    pm = jnp.maximum(
        jnp.dot(h, wd_ref[...], preferred_element_type=jnp.float32) + bd_ref[...], 0.0)
    pm_ref[...] = pm[:, 0:1]
    for k in range(_NUM_PRED):
        pf_ref[k] = jnp.tanh(
            jnp.dot(h, wf_ref[:, pl.ds(k * _D, _D)],
                    preferred_element_type=jnp.float32)
            + bf_ref[:, pl.ds(k * _D, _D)])


_BN = 1000


def _tc_dense(hpre, aggout, deg, noise, WnA, WnB, Wd8, bd8, W_f, bf2):
    return pl.pallas_call(
        _tc_body,
        grid=(_N // _BN,),
        in_specs=[
            pl.BlockSpec((_BN, _D), lambda i: (i, 0)),
            pl.BlockSpec((1, _BN, _DH), lambda i: (0, i, 0)),
            pl.BlockSpec((1, _BN, _DH), lambda i: (1, i, 0)),
            pl.BlockSpec((_BN, 1), lambda i: (i, 0)),
            pl.BlockSpec((_BN, _D), lambda i: (i, 0)),
            pl.BlockSpec((_DH, _D), lambda i: (0, 0)),
            pl.BlockSpec((_DH, _D), lambda i: (0, 0)),
            pl.BlockSpec((_D, 8), lambda i: (0, 0)),
            pl.BlockSpec((1, 8), lambda i: (0, 0)),
            pl.BlockSpec((_D, _NUM_PRED * _D), lambda i: (0, 0)),
            pl.BlockSpec((1, _NUM_PRED * _D), lambda i: (0, 0)),
        ],
        out_specs=[
            pl.BlockSpec((_BN, 1), lambda i: (i, 0)),
            pl.BlockSpec((_NUM_PRED, _BN, _D), lambda i: (0, i, 0)),
        ],
        out_shape=[
            jax.ShapeDtypeStruct((_N, 1), jnp.float32),
            jax.ShapeDtypeStruct((_NUM_PRED, _N, _D), jnp.float32),
        ],
    )(hpre, aggout, aggout, deg, noise, WnA, WnB, Wd8, bd8, W_f, bf2)


def kernel(x, edge_index, noise, W_self, W_neigh, b_enc, W_d, b_d, W_f, b_f):
    src = edge_index[0].astype(jnp.int32)
    dst = edge_index[1].astype(jnp.int32)
    xp = x.reshape(2 * _N, _DH)
    npad = _EPAD - _E
    src_p = jnp.concatenate([src, jnp.zeros((npad,), jnp.int32)])
    dst_p = jnp.concatenate([dst, jnp.full((npad,), _N, jnp.int32)])
    gidx = jnp.concatenate([2 * src_p, 2 * src_p + 1])

    be2 = b_enc.reshape(1, _D)
    hpre = _tc_pre(x, W_self, be2)
    aggout, deg16 = _sc_agg(xp, gidx, dst_p)

    Wd8 = jnp.pad(W_d, ((0, 0), (0, 7)))
    bd8 = jnp.pad(b_d, (0, 7)).reshape(1, 8)
    bf2 = b_f.reshape(1, _NUM_PRED * _D)
    WnA = W_neigh[:_DH]
    WnB = W_neigh[_DH:]
    deg2 = deg16.sum(axis=0)[:_N].reshape(_N, 1)

    pm, pf5 = _tc_dense(hpre, aggout, deg2, noise, WnA, WnB,
                        Wd8, bd8, W_f, bf2)
    return (pm, pf5.transpose(1, 0, 2))


# speedup vs baseline: 1.2085x; 1.0033x over previous
import functools

import jax
import jax.numpy as jnp
from jax import lax
from jax.experimental import pallas as pl
from jax.experimental.pallas import tpu as pltpu
from jax.experimental.pallas import tpu_sc as plsc

_N = 10000
_D = 256
_E = 160000
_NUM_PRED = 5
_DH = 128
_NP = 10112
_NS = 16
_CHUNK = 64
_ET = 10240
_NCHT = _ET // _CHUNK
_NCH = _NS * _NCHT
_EPAD = _NS * _ET
_RPT = _NP // _NS


def _take16(v, idx):
    return lax.gather(
        v, idx.reshape(16, 1),
        lax.GatherDimensionNumbers(offset_dims=(), collapsed_slice_dims=(0,),
                                   start_index_map=(0,)),
        slice_sizes=(1,), mode=lax.GatherScatterMode.PROMISE_IN_BOUNDS)


def _deg_count(idx_ref, deg_ref):
    iota = lax.iota(jnp.int32, 16)
    prev_i = jnp.maximum(iota - 1, 0)
    next_i = jnp.minimum(iota + 1, 15)
    for g in range(_CHUNK // 16):
        d = idx_ref[pl.ds(g * 16, 16)]
        d, _ = plsc.sort_key_val(d, d)
        dprev = _take16(d, prev_i)
        dnext = _take16(d, next_i)
        starts = (iota == 0) | (d != dprev)
        ends = (iota == 15) | (d != dnext)
        st = plsc.cummax(jnp.where(starts, iota, 0))
        cnt = (iota - st + 1).astype(jnp.float32)
        plsc.addupdate_scatter(deg_ref, [d], cnt, mask=ends)


def _sc_agg(xp, gidx, didx):
    mesh = plsc.VectorSubcoreMesh(core_axis_name="c", subcore_axis_name="s")

    @functools.partial(
        pl.kernel,
        out_type=[jax.ShapeDtypeStruct((2, _NP, _DH), jnp.float32),
                  jax.ShapeDtypeStruct((2 * _NS, _NP), jnp.float32)],
        mesh=mesh,
        scratch_types=[
            [pltpu.VMEM((_CHUNK,), jnp.int32) for _ in range(8)],
            [pltpu.VMEM((_CHUNK,), jnp.int32) for _ in range(8)],
            [pltpu.VMEM((_CHUNK, _DH), jnp.float32) for _ in range(4)],
            pltpu.VMEM((_NP,), jnp.float32),
            pltpu.VMEM_SHARED((_NP, _DH), jnp.float32),
            [pltpu.SemaphoreType.DMA for _ in range(8)],
            [pltpu.SemaphoreType.DMA for _ in range(8)],
            [pltpu.SemaphoreType.DMA for _ in range(4)],
            [pltpu.SemaphoreType.DMA for _ in range(4)],
        ],
        compiler_params=pltpu.CompilerParams(use_tc_tiling_on_sc=True, needs_layout_passes=False),
    )
    def k(xp_hbm, gidx_hbm, didx_hbm, out_hbm, deg_hbm,
          gbuf, dbuf, rows, deg_v, agg_sh, isem, jsem, gsem, ssem):
        c = lax.axis_index("c")
        s = lax.axis_index("s")
        r0 = s * _RPT

        def zero_rows(i, carry):
            for j in range(_DH // 16):
                rows[0][i, pl.ds(j * 16, 16)] = jnp.zeros((16,), jnp.float32)
            return carry

        lax.fori_loop(0, _CHUNK, zero_rows, 0)
        for q in range(_RPT // _CHUNK):
            pltpu.sync_copy(rows[0], agg_sh.at[pl.ds(r0 + q * _CHUNK, _CHUNK)])
        _REM = _RPT - (_RPT // _CHUNK) * _CHUNK
        if _REM:
            pltpu.sync_copy(
                rows[0].at[pl.ds(0, _REM)],
                agg_sh.at[pl.ds(r0 + _RPT - _REM, _REM)])

        def zero_deg(i, carry):
            deg_v[pl.ds(i * 16, 16)] = jnp.zeros((16,), jnp.float32)
            return carry

        lax.fori_loop(0, _NP // 16, zero_deg, 0)

        ebase = (s * _NCHT) * _CHUNK
        gbase = c * _EPAD + ebase

        def load_idx(i, a):
            pltpu.async_copy(
                gidx_hbm.at[pl.ds(gbase + i * _CHUNK, _CHUNK)], gbuf[a], isem[a])
            pltpu.async_copy(
                didx_hbm.at[pl.ds(ebase + i * _CHUNK, _CHUNK)], dbuf[a], jsem[a])

        def wait_idx(i, a):
            pltpu.make_async_copy(
                gidx_hbm.at[pl.ds(gbase + i * _CHUNK, _CHUNK)], gbuf[a], isem[a]).wait()
            pltpu.make_async_copy(
                didx_hbm.at[pl.ds(ebase + i * _CHUNK, _CHUNK)], dbuf[a], jsem[a]).wait()

        on_c0 = c == 0
        plsc.subcore_barrier()
        for a in range(6):
            load_idx(a, a)
        for b in range(2):
            wait_idx(b, b)
            pltpu.async_copy(xp_hbm.at[gbuf[b]], rows[b], gsem[b])

        def outer(g, carry):
            for a in range(8):
                i = g * 8 + a
                b = a % 4
                pltpu.make_async_copy(
                    xp_hbm.at[gbuf[a]], rows[b], gsem[b]).wait()
                pltpu.async_copy(
                    rows[b], agg_sh.at[dbuf[a]], ssem[b], add=True)
                b2 = (b + 2) % 4
                a2 = (a + 2) % 8
                a6 = (a + 6) % 8

                @pl.when(i >= 2)
                def _():
                    pltpu.make_async_copy(
                        rows[b2], agg_sh.at[dbuf[a6]], ssem[b2]).wait()

                @pl.when(i + 6 < _NCHT)
                def _():
                    load_idx(i + 6, a6)

                @pl.when(i + 2 < _NCHT)
                def _():
                    wait_idx(i + 2, a2)
                    pltpu.async_copy(xp_hbm.at[gbuf[a2]], rows[b2], gsem[b2])

                if a % 2 == 0:
                    @pl.when(on_c0)
                    def _():
                        _deg_count(dbuf[a], deg_v)
                else:
                    @pl.when(jnp.logical_not(on_c0))
                    def _():
                        _deg_count(dbuf[a], deg_v)
            return carry

        lax.fori_loop(0, _NCHT // 8, outer, 0)
        for j in (_NCHT - 2, _NCHT - 1):
            pltpu.make_async_copy(
                rows[j % 4], agg_sh.at[dbuf[j % 8]], ssem[j % 4]).wait()
        plsc.subcore_barrier()
        pltpu.sync_copy(agg_sh.at[pl.ds(r0, _RPT)], out_hbm.at[c, pl.ds(r0, _RPT)])

        pltpu.sync_copy(deg_v, deg_hbm.at[c * _NS + s])

    return k(xp, gidx, didx)


def _tc_pre_body(x_ref, ws_ref, be_ref, hp_ref):
    hp_ref[...] = (jnp.dot(x_ref[...], ws_ref[...],
                           preferred_element_type=jnp.float32) + be_ref[...])


def _tc_pre(x, W_self, be2):
    return pl.pallas_call(
        _tc_pre_body,
        grid=(_N // _BN,),
        in_specs=[
            pl.BlockSpec((_BN, _D), lambda i: (i, 0)),
            pl.BlockSpec((_D, _D), lambda i: (0, 0)),
            pl.BlockSpec((1, _D), lambda i: (0, 0)),
        ],
        out_specs=[pl.BlockSpec((_BN, _D), lambda i: (i, 0))],
        out_shape=[jax.ShapeDtypeStruct((_N, _D), jnp.float32)],
    )(x, W_self, be2)[0]


def _tc_body(hp_ref, aA_ref, aB_ref, dg_ref, nz_ref, wnA_ref, wnB_ref,
             wd_ref, bd_ref, wf_ref, bf_ref, pm_ref, pf_ref):
    d = jnp.maximum(dg_ref[...], 1.0)
    aA = aA_ref[0] / d
    aB = aB_ref[0] / d
    h = hp_ref[...]
    h = h + jnp.dot(aA, wnA_ref[...], preferred_element_type=jnp.float32)
    h = h + jnp.dot(aB, wnB_ref[...], preferred_element_type=jnp.float32)
    h = jnp.maximum(h, 0.0) + nz_ref[...]
    pm = jnp.maximum(
        jnp.dot(h, wd_ref[...], preferred_element_type=jnp.float32) + bd_ref[...], 0.0)
    pm_ref[...] = pm[:, 0:1]
    for k in range(_NUM_PRED):
        pf_ref[k] = jnp.tanh(
            jnp.dot(h, wf_ref[:, pl.ds(k * _D, _D)],
                    preferred_element_type=jnp.float32)
            + bf_ref[:, pl.ds(k * _D, _D)])


_BN = 2000


def _tc_dense(hpre, aggout, deg, noise, WnA, WnB, Wd8, bd8, W_f, bf2):
    return pl.pallas_call(
        _tc_body,
        grid=(_N // _BN,),
        in_specs=[
            pl.BlockSpec((_BN, _D), lambda i: (i, 0)),
            pl.BlockSpec((1, _BN, _DH), lambda i: (0, i, 0)),
            pl.BlockSpec((1, _BN, _DH), lambda i: (1, i, 0)),
            pl.BlockSpec((_BN, 1), lambda i: (i, 0)),
            pl.BlockSpec((_BN, _D), lambda i: (i, 0)),
            pl.BlockSpec((_DH, _D), lambda i: (0, 0)),
            pl.BlockSpec((_DH, _D), lambda i: (0, 0)),
            pl.BlockSpec((_D, 8), lambda i: (0, 0)),
            pl.BlockSpec((1, 8), lambda i: (0, 0)),
            pl.BlockSpec((_D, _NUM_PRED * _D), lambda i: (0, 0)),
            pl.BlockSpec((1, _NUM_PRED * _D), lambda i: (0, 0)),
        ],
        out_specs=[
            pl.BlockSpec((_BN, 1), lambda i: (i, 0)),
            pl.BlockSpec((_NUM_PRED, _BN, _D), lambda i: (0, i, 0)),
        ],
        out_shape=[
            jax.ShapeDtypeStruct((_N, 1), jnp.float32),
            jax.ShapeDtypeStruct((_NUM_PRED, _N, _D), jnp.float32),
        ],
    )(hpre, aggout, aggout, deg, noise, WnA, WnB, Wd8, bd8, W_f, bf2)


def kernel(x, edge_index, noise, W_self, W_neigh, b_enc, W_d, b_d, W_f, b_f):
    src = edge_index[0].astype(jnp.int32)
    dst = edge_index[1].astype(jnp.int32)
    xp = x.reshape(2 * _N, _DH)
    npad = _EPAD - _E
    src_p = jnp.concatenate([src, jnp.zeros((npad,), jnp.int32)])
    dst_p = jnp.concatenate([dst, jnp.full((npad,), _N, jnp.int32)])
    gidx = jnp.concatenate([2 * src_p, 2 * src_p + 1])

    be2 = b_enc.reshape(1, _D)
    hpre = _tc_pre(x, W_self, be2)
    aggout, deg16 = _sc_agg(xp, gidx, dst_p)

    Wd8 = jnp.pad(W_d, ((0, 0), (0, 7)))
    bd8 = jnp.pad(b_d, (0, 7)).reshape(1, 8)
    bf2 = b_f.reshape(1, _NUM_PRED * _D)
    WnA = W_neigh[:_DH]
    WnB = W_neigh[_DH:]
    deg2 = deg16.sum(axis=0)[:_N].reshape(_N, 1)

    pm, pf5 = _tc_dense(hpre, aggout, deg2, noise, WnA, WnB,
                        Wd8, bd8, W_f, bf2)
    return (pm, pf5.transpose(1, 0, 2))
